# baseline (speedup 1.0000x reference)
import functools

import jax
import jax.numpy as jnp
from jax import lax
from jax.experimental import pallas as pl
from jax.experimental.pallas import tpu as pltpu
from jax.experimental.pallas import tpu_sc as plsc

_NUM_CORES = 2
_NUM_SUBCORES = 16
_NUM_WORKERS = _NUM_CORES * _NUM_SUBCORES
_C = 128
_L = 16
_D = 100
_PL = 7
_DO = 300


def _lane_gather(v, idx):
    dnums = lax.GatherDimensionNumbers(
        offset_dims=(), collapsed_slice_dims=(0,), start_index_map=(0,))
    return lax.gather(v, idx[:, None], dnums, (1,),
                      mode=lax.GatherScatterMode.PROMISE_IN_BOUNDS)


def _embed_lookup(words, tags, lemmas, word_table, tag_table, lemma_table):
    n = words.shape[0]
    per_w = n // _NUM_WORKERS
    n_chunks = per_w // _C
    assert per_w * _NUM_WORKERS == n and n_chunks * _C == per_w

    mesh = plsc.VectorSubcoreMesh(core_axis_name="c", subcore_axis_name="s")

    @functools.partial(
        pl.kernel,
        out_type=jax.ShapeDtypeStruct((1, n * _DO), jnp.float32),
        mesh=mesh,
        compiler_params=pltpu.CompilerParams(use_tc_tiling_on_sc=False),
        scratch_types=[
            pltpu.VMEM((per_w,), jnp.int32),
            pltpu.VMEM((per_w,), jnp.int32),
            pltpu.VMEM((per_w,), jnp.int32),
            pltpu.VMEM((3, _PL + 1, _C, _L), jnp.float32),
            pltpu.VMEM((3 * _PL, 1, _C), jnp.int32),
            pltpu.VMEM((1, _C * _DO), jnp.float32),
            pltpu.SemaphoreType.DMA,
        ],
    )
    def k(words_h, tags_h, lemmas_h, wt_h, tt_h, lt_h, out_h,
          iw, it, il, gbuf, pidx, comb, gsem):
        wid = lax.axis_index("s") * _NUM_CORES + lax.axis_index("c")
        base = wid * per_w
        pltpu.sync_copy(words_h.at[pl.ds(base, per_w)], iw)
        pltpu.sync_copy(tags_h.at[pl.ds(base, per_w)], it)
        pltpu.sync_copy(lemmas_h.at[pl.ds(base, per_w)], il)

        lane = lax.iota(jnp.int32, _L)
        bands = ((0, iw, wt_h), (1, it, tt_h), (2, il, lt_h))

        @pl.loop(0, n_chunks)
        def body(c):
            off = c * _C

            for b, ivb, _ in bands:
                for g in range(_C // _L):
                    v = ivb[pl.ds(off + g * _L, _L)]
                    j0 = 6 * v + jnp.right_shift(v, 2)
                    for r in range(_PL):
                        pidx[b * _PL + r, 0, pl.ds(g * _L, _L)] = j0 + r

            descs = []
            for b, _, tbl in bands:
                for r in range(_PL):
                    descs.append(pltpu.async_copy(
                        tbl.at[pidx.at[b * _PL + r, 0]],
                        gbuf.at[b, r], gsem))
            for d in descs:
                d.wait()

            @pl.loop(0, _C // _L)
            def asm(g):
                for b, ivb, _ in bands:
                    v16 = ivb[pl.ds(off + g * _L, _L)]
                    sv = jnp.left_shift(jnp.bitwise_and(v16, 3), 2)
                    for j in range(_L):
                        kk = g * _L + j
                        s = sv[j]
                        rotidx = jnp.bitwise_and(lane + s, _L - 1)
                        keep = lane < (_L - s)
                        rv = []
                        for r in range(_PL):
                            vr = gbuf[b, r, kk, :]
                            rv.append(_lane_gather(vr, rotidx))
                        dst0 = kk * _DO + b * _D
                        for m in range(_D // _L):
                            grp = jnp.where(keep, rv[m], rv[m + 1])
                            comb[0, pl.ds(dst0 + m * _L, _L)] = grp
                        s2 = jnp.bitwise_and(s + 4, _L - 1)
                        p0 = jnp.right_shift(s + 84, 4)
                        rot2 = jnp.bitwise_and(lane + s2, _L - 1)
                        keep2 = lane < (_L - s2)
                        va = gbuf[b, p0, kk, :]
                        vb = gbuf[b, jnp.minimum(p0 + 1, _PL - 1), kk, :]
                        ra = _lane_gather(va, rot2)
                        rb = _lane_gather(vb, rot2)
                        grp = jnp.where(keep2, ra, rb)
                        comb[0, pl.ds(dst0 + _D - _L, _L)] = grp

            pltpu.sync_copy(
                comb, out_h.at[:, pl.ds((base + off) * _DO, _C * _DO)])

    return k(words, tags, lemmas, word_table, tag_table, lemma_table)


def kernel(words, tags, lemmas, word_table, tag_table, lemma_table):
    b, l = words.shape
    d = word_table.shape[1] + tag_table.shape[1] + lemma_table.shape[1]
    out = _embed_lookup(
        words.reshape(-1), tags.reshape(-1), lemmas.reshape(-1),
        word_table.reshape(-1, _L), tag_table.reshape(-1, _L),
        lemma_table.reshape(-1, _L))
    return out.reshape(b, l, d)


# speedup vs baseline: 1.0372x; 1.0372x over previous
import functools

import jax
import jax.numpy as jnp
from jax import lax
from jax.experimental import pallas as pl
from jax.experimental.pallas import tpu as pltpu
from jax.experimental.pallas import tpu_sc as plsc

_NUM_CORES = 2
_NUM_SUBCORES = 16
_NUM_WORKERS = _NUM_CORES * _NUM_SUBCORES
_C = 80
_L = 16
_D = 100
_PL = 7
_DO = 300


def _embed_lookup(idx_all, tbl_all, n, tbases):
    per_w = n // _NUM_WORKERS
    n_chunks = per_w // _C
    n_pairs = n_chunks // 2
    assert per_w * _NUM_WORKERS == n and n_chunks * _C == per_w
    assert n_pairs * 2 == n_chunks and n_pairs >= 2

    mesh = plsc.VectorSubcoreMesh(core_axis_name="c", subcore_axis_name="s")

    @functools.partial(
        pl.kernel,
        out_type=jax.ShapeDtypeStruct((n * _DO,), jnp.float32),
        mesh=mesh,
        compiler_params=pltpu.CompilerParams(
            use_tc_tiling_on_sc=False, needs_layout_passes=False),
        scratch_types=[
            pltpu.VMEM((2, 3, _C), jnp.int32),
            pltpu.VMEM((2, 3, _PL, _C, _L), jnp.float32),
            pltpu.VMEM((2, 3 * _PL, 1, _C), jnp.int32),
            pltpu.VMEM((2, _C * _DO), jnp.float32),
            pltpu.SemaphoreType.DMA,
            pltpu.SemaphoreType.DMA,
        ],
    )
    def k(idx_h, tbl_h, out_h, ibuf, gbuf, pidx, comb, semA, semB):
        wid = lax.axis_index("s") * _NUM_CORES + lax.axis_index("c")
        base = wid * per_w
        lane = lax.iota(jnp.int32, _L)
        sems = (semA, semB)

        def fire(c, sl):
            off = base + c * _C
            for b in range(3):
                pltpu.sync_copy(
                    idx_h.at[pl.ds(b * n + off, _C)], ibuf.at[sl, b])
            for b in range(3):
                for g in range(_C // _L):
                    v = ibuf[sl, b, pl.ds(g * _L, _L)]
                    j0 = 6 * v + jnp.right_shift(v, 2) + tbases[b]
                    for r in range(_PL):
                        pidx[sl, b * _PL + r, 0, pl.ds(g * _L, _L)] = j0 + r
            for b in range(3):
                for r in range(_PL):
                    pltpu.async_copy(
                        tbl_h.at[pidx.at[sl, b * _PL + r, 0]],
                        gbuf.at[sl, b, r], sems[sl])

        def drain(sl):
            for b in range(3):
                for r in range(_PL):
                    pltpu.make_async_copy(
                        tbl_h.at[pidx.at[sl, b * _PL + r, 0]],
                        gbuf.at[sl, b, r], sems[sl]).wait()

        def asm_store(c, sl):
            cref = comb.at[sl]

            @pl.loop(0, _C // _L)
            def asm(g):
                for b in range(3):
                    v16 = ibuf[sl, b, pl.ds(g * _L, _L)]
                    sv = jnp.left_shift(jnp.bitwise_and(v16, 3), 2)
                    for j in range(_L):
                        kk = g * _L + j
                        s = sv[j]
                        ib = (kk * _DO + b * _D - s) + lane
                        m0 = lane >= s
                        m6 = lane < (s + 4)
                        for r in range(_PL):
                            x = gbuf[sl, b, r, kk, :]
                            idx = ib + (r * _L)
                            if r == 0:
                                plsc.store_scatter(cref, [idx], x, mask=m0)
                            elif r == _PL - 1:
                                plsc.store_scatter(cref, [idx], x, mask=m6)
                            else:
                                plsc.store_scatter(cref, [idx], x)

            pltpu.sync_copy(
                cref, out_h.at[pl.ds((base + c * _C) * _DO, _C * _DO)])

        fire(0, 0)

        @pl.loop(0, n_pairs - 1)
        def pair(h):
            c0 = 2 * h
            fire(c0 + 1, 1)
            drain(0)
            asm_store(c0, 0)
            fire(c0 + 2, 0)
            drain(1)
            asm_store(c0 + 1, 1)

        c0 = n_chunks - 2
        fire(c0 + 1, 1)
        drain(0)
        asm_store(c0, 0)
        drain(1)
        asm_store(c0 + 1, 1)

    return k(idx_all, tbl_all)


def kernel(words, tags, lemmas, word_table, tag_table, lemma_table):
    b, l = words.shape
    d = word_table.shape[1] + tag_table.shape[1] + lemma_table.shape[1]
    n = b * l
    nw = word_table.shape[0] * word_table.shape[1]
    nt = tag_table.shape[0] * tag_table.shape[1]
    tbases = (0, nw // _L, (nw + nt) // _L)
    idx_all = jnp.concatenate(
        [words.reshape(-1), tags.reshape(-1), lemmas.reshape(-1)])
    tbl_all = jnp.concatenate(
        [word_table.reshape(-1), tag_table.reshape(-1),
         lemma_table.reshape(-1)]).reshape(-1, _L)
    out = _embed_lookup(idx_all, tbl_all, n, tbases)
    return out.reshape(b, l, d)
